# baseline (speedup 1.0000x reference)
import jax
import jax.numpy as jnp
from jax import lax
from jax.experimental import pallas as pl
from jax.experimental.pallas import tpu as pltpu
from jax.experimental.pallas import tpu_sc as plsc

SEQ_LEN = 4096
BATCH = 4
D_MODEL = 1024
LANES = 16

NC, NS = 2, 16
NW = NC * NS
LPW = SEQ_LEN // NW
CL = 4
NCHUNK = LPW // CL
CROWS = CL * 32
NBUF = 2
NG = NCHUNK // NBUF

XROWS = SEQ_LEN * 32
PEROWS = SEQ_LEN * 8

_GDN = lax.GatherDimensionNumbers(
    offset_dims=(), collapsed_slice_dims=(0,), start_index_map=(0,))


def _vgather(v, idx):
    return lax.gather(v, idx[:, None], dimension_numbers=_GDN,
                      slice_sizes=(1,),
                      mode=lax.GatherScatterMode.PROMISE_IN_BOUNDS)


def _body(x_hbm, idx_hbm, pe_hbm, out_hbm, idxq, lbuf, xbuf,
          in_s0, in_s1, out_s0, out_s1, ga_sem):
    in_sems = (in_s0, in_s1)
    out_sems = (out_s0, out_s1)
    wid = lax.axis_index("s") * NC + lax.axis_index("c")
    xbase = wid * (LPW * 32)
    pltpu.sync_copy(idx_hbm.at[pl.ds(wid * BATCH, BATCH)], idxq)

    iota = lax.iota(jnp.int32, LANES)
    perm_lo = lax.bitwise_and(iota, 3)
    dt_hi = lax.shift_right_logical(iota, 2)

    def start_in(b, c):
        pltpu.async_copy(x_hbm.at[pl.ds(xbase + c * CROWS, CROWS)],
                         xbuf.at[b], in_sems[b])
        row = lax.shift_right_logical(c, 3)
        col = lax.bitwise_and(c, 7) * LANES
        raw = idxq[row, pl.ds(col, LANES)]
        for i in range(8):
            perm = perm_lo + 4 * (i // 2)
            dt = dt_hi + 4 * (i % 2)
            lbuf[b, pl.ds(i * LANES, LANES)] = _vgather(raw, perm) * 8 + dt

    def wait_in(b):
        pltpu.make_async_copy(x_hbm.at[pl.ds(0, CROWS)], xbuf.at[b],
                              in_sems[b]).wait()

    def start_out(b, c):
        pltpu.async_copy(xbuf.at[b], out_hbm.at[pl.ds(xbase + c * CROWS, CROWS)],
                         out_sems[b])

    def wait_out(b):
        pltpu.make_async_copy(xbuf.at[b], out_hbm.at[pl.ds(0, CROWS)],
                              out_sems[b]).wait()

    def group(g, carry):
        for b in range(NBUF):
            c = g * NBUF + b
            pl.when(g >= 1)(lambda: wait_out(b))
            start_in(b, c)
            wait_in(b)
            pltpu.async_copy(pe_hbm.at[lbuf.at[b]], xbuf.at[b], ga_sem,
                             add=True)
            pltpu.make_async_copy(x_hbm.at[pl.ds(0, CROWS)], xbuf.at[b],
                                  ga_sem).wait()
            start_out(b, c)
        return carry

    lax.fori_loop(0, NG, group, 0)
    for b in range(NBUF):
        wait_out(b)


@jax.jit
def _sc_add_gather(xv, idxv, pev):
    mesh = plsc.VectorSubcoreMesh(
        core_axis_name="c", subcore_axis_name="s",
        num_cores=NC, num_subcores=NS,
    )
    return pl.kernel(
        _body,
        out_type=jax.ShapeDtypeStruct((XROWS, 128), jnp.float32),
        mesh=mesh,
        scratch_types=[
            pltpu.VMEM((BATCH, 128), jnp.int32),
            pltpu.VMEM((NBUF, CROWS), jnp.int32),
            pltpu.VMEM((NBUF, CROWS, 128), jnp.float32),
            pltpu.SemaphoreType.DMA,
            pltpu.SemaphoreType.DMA,
            pltpu.SemaphoreType.DMA,
            pltpu.SemaphoreType.DMA,
            pltpu.SemaphoreType.DMA,
        ],
    )(xv, idxv, pev)


def kernel(x, index, pe):
    xv = jnp.transpose(x.reshape(SEQ_LEN, BATCH, 8, 128),
                       (0, 2, 1, 3)).reshape(XROWS, 128)
    pev = pe.reshape(PEROWS, 128)
    idxv = index.astype(jnp.int32).T.reshape(128, 128)
    o = _sc_add_gather(xv, idxv, pev)
    return jnp.transpose(o.reshape(SEQ_LEN, 8, BATCH, 128),
                         (0, 2, 1, 3)).reshape(SEQ_LEN, BATCH, D_MODEL)


# speedup vs baseline: 1.3694x; 1.3694x over previous
import jax
import jax.numpy as jnp
from jax import lax
from jax.experimental import pallas as pl
from jax.experimental.pallas import tpu as pltpu
from jax.experimental.pallas import tpu_sc as plsc

SEQ_LEN = 4096
BATCH = 4
D_MODEL = 1024
LANES = 16

NC, NS = 2, 16
NW = NC * NS
LPW = SEQ_LEN // NW
CL = 4
NCHUNK = LPW // CL
CROWS = CL * 32
NBUF = 4

XROWS = SEQ_LEN * 32
PEROWS = SEQ_LEN * 8

_GDN = lax.GatherDimensionNumbers(
    offset_dims=(), collapsed_slice_dims=(0,), start_index_map=(0,))


def _vgather(v, idx):
    return lax.gather(v, idx[:, None], dimension_numbers=_GDN,
                      slice_sizes=(1,),
                      mode=lax.GatherScatterMode.PROMISE_IN_BOUNDS)


def _body(x_hbm, idx_hbm, pe_hbm, out_hbm, idxq, lbuf, xbuf,
          semx, semga, semout):
    wid = lax.axis_index("s") * NC + lax.axis_index("c")
    xbase = wid * (LPW * 32)
    pltpu.sync_copy(idx_hbm.at[pl.ds(wid * BATCH, BATCH)], idxq)

    iota = lax.iota(jnp.int32, LANES)
    perm_lo = lax.bitwise_and(iota, 3)
    dt_hi = lax.shift_right_logical(iota, 2)

    def start_x(b, c):
        pltpu.async_copy(x_hbm.at[pl.ds(xbase + c * CROWS, CROWS)],
                         xbuf.at[b], semx.at[b])

    def wait_x(b):
        pltpu.make_async_copy(x_hbm.at[pl.ds(0, CROWS)], xbuf.at[b],
                              semx.at[b]).wait()

    def start_ga(b, c):
        row = lax.shift_right_logical(c, 3)
        col = lax.bitwise_and(c, 7) * LANES
        raw = idxq[row, pl.ds(col, LANES)]
        for i in range(8):
            perm = perm_lo + 4 * (i // 2)
            dt = dt_hi + 4 * (i % 2)
            lbuf[b, pl.ds(i * LANES, LANES)] = _vgather(raw, perm) * 8 + dt
        pltpu.async_copy(pe_hbm.at[lbuf.at[b]], xbuf.at[b], semga.at[b],
                         add=True)

    def wait_ga(b):
        pltpu.make_async_copy(x_hbm.at[pl.ds(0, CROWS)], xbuf.at[b],
                              semga.at[b]).wait()

    def start_out(b, c):
        pltpu.async_copy(xbuf.at[b], out_hbm.at[pl.ds(xbase + c * CROWS, CROWS)],
                         semout.at[b])

    def wait_out(b):
        pltpu.make_async_copy(xbuf.at[b], out_hbm.at[pl.ds(0, CROWS)],
                              semout.at[b]).wait()

    for k in range(NBUF - 2):
        start_x(k, k)

    def group(g, carry):
        for b in range(NBUF):
            c = g * NBUF + b
            wait_x(b)
            start_ga(b, c)
            cf = c + NBUF - 2
            bpre = (b + NBUF - 2) % NBUF

            @pl.when(cf < NCHUNK)
            def _():
                pl.when(cf >= NBUF)(lambda: wait_out(bpre))
                start_x(bpre, cf)

            bm1 = (b + NBUF - 1) % NBUF

            @pl.when(c >= 1)
            def _():
                wait_ga(bm1)
                start_out(bm1, c - 1)
        return carry

    lax.fori_loop(0, NCHUNK // NBUF, group, 0)

    blast = (NCHUNK - 1) % NBUF
    wait_ga(blast)
    start_out(blast, NCHUNK - 1)
    for k in range(NCHUNK - NBUF, NCHUNK):
        wait_out(k % NBUF)


@jax.jit
def _sc_add_gather(xv, idxv, pev):
    mesh = plsc.VectorSubcoreMesh(
        core_axis_name="c", subcore_axis_name="s",
        num_cores=NC, num_subcores=NS,
    )
    return pl.kernel(
        _body,
        out_type=jax.ShapeDtypeStruct((XROWS, 128), jnp.float32),
        mesh=mesh,
        scratch_types=[
            pltpu.VMEM((BATCH, 128), jnp.int32),
            pltpu.VMEM((NBUF, CROWS), jnp.int32),
            pltpu.VMEM((NBUF, CROWS, 128), jnp.float32),
            pltpu.SemaphoreType.DMA((NBUF,)),
            pltpu.SemaphoreType.DMA((NBUF,)),
            pltpu.SemaphoreType.DMA((NBUF,)),
        ],
    )(xv, idxv, pev)


def kernel(x, index, pe):
    xv = jnp.transpose(x.reshape(SEQ_LEN, BATCH, 8, 128),
                       (0, 2, 1, 3)).reshape(XROWS, 128)
    pev = pe.reshape(PEROWS, 128)
    idxv = index.astype(jnp.int32).T.reshape(128, 128)
    o = _sc_add_gather(xv, idxv, pev)
    return jnp.transpose(o.reshape(SEQ_LEN, 8, BATCH, 128),
                         (0, 2, 1, 3)).reshape(SEQ_LEN, BATCH, D_MODEL)


# speedup vs baseline: 1.3787x; 1.0067x over previous
import jax
import jax.numpy as jnp
from jax import lax
from jax.experimental import pallas as pl
from jax.experimental.pallas import tpu as pltpu
from jax.experimental.pallas import tpu_sc as plsc

SEQ_LEN = 4096
BATCH = 4
D_MODEL = 1024
LANES = 16

NC, NS = 2, 16
NW = NC * NS
LPW = SEQ_LEN // NW
CL = 4
NCHUNK = LPW // CL
CROWS = CL * 32
NBUF = 4

XROWS = SEQ_LEN * 32
PEROWS = SEQ_LEN * 8

_GDN = lax.GatherDimensionNumbers(
    offset_dims=(), collapsed_slice_dims=(0,), start_index_map=(0,))


def _vgather(v, idx):
    return lax.gather(v, idx[:, None], dimension_numbers=_GDN,
                      slice_sizes=(1,),
                      mode=lax.GatherScatterMode.PROMISE_IN_BOUNDS)


def _body(x_hbm, idx_hbm, pe_hbm, out_hbm, idxq, lbuf, xbuf,
          semx, semga, semout):
    wid = lax.axis_index("s") * NC + lax.axis_index("c")
    xbase = wid * (LPW * 32)

    iota = lax.iota(jnp.int32, LANES)
    perm_lo = lax.bitwise_and(iota, 3)
    dt_hi = lax.shift_right_logical(iota, 2)

    def start_x(b, c):
        pltpu.async_copy(x_hbm.at[pl.ds(xbase + c * CROWS, CROWS)],
                         xbuf.at[b], semx.at[b])

    def wait_x(b):
        pltpu.make_async_copy(x_hbm.at[pl.ds(0, CROWS)], xbuf.at[b],
                              semx.at[b]).wait()

    def start_ga(b, c):
        row = lax.shift_right_logical(c, 3)
        col = lax.bitwise_and(c, 7) * LANES
        raw = idxq[row, pl.ds(col, LANES)]
        for i in range(8):
            perm = perm_lo + 4 * (i // 2)
            dt = dt_hi + 4 * (i % 2)
            lbuf[b, pl.ds(i * LANES, LANES)] = _vgather(raw, perm) * 8 + dt
        pltpu.async_copy(pe_hbm.at[lbuf.at[b]], xbuf.at[b], semga.at[b],
                         add=True)

    def wait_ga(b):
        pltpu.make_async_copy(x_hbm.at[pl.ds(0, CROWS)], xbuf.at[b],
                              semga.at[b]).wait()

    def start_out(b, c):
        pltpu.async_copy(xbuf.at[b], out_hbm.at[pl.ds(xbase + c * CROWS, CROWS)],
                         semout.at[b])

    def wait_out(b):
        pltpu.make_async_copy(xbuf.at[b], out_hbm.at[pl.ds(0, CROWS)],
                              semout.at[b]).wait()

    for k in range(NBUF - 2):
        start_x(k, k)
    pltpu.sync_copy(idx_hbm.at[pl.ds(wid * BATCH, BATCH)], idxq)

    def group(g, carry):
        for b in range(NBUF):
            c = g * NBUF + b
            wait_x(b)
            start_ga(b, c)
            cf = c + NBUF - 2
            bpre = (b + NBUF - 2) % NBUF

            @pl.when(cf < NCHUNK)
            def _():
                pl.when(cf >= NBUF)(lambda: wait_out(bpre))
                start_x(bpre, cf)

            bm1 = (b + NBUF - 1) % NBUF

            @pl.when(c >= 1)
            def _():
                wait_ga(bm1)
                start_out(bm1, c - 1)
        return carry

    lax.fori_loop(0, NCHUNK // NBUF, group, 0)

    blast = (NCHUNK - 1) % NBUF
    wait_ga(blast)
    start_out(blast, NCHUNK - 1)
    for k in range(NCHUNK - NBUF, NCHUNK):
        wait_out(k % NBUF)


@jax.jit
def _sc_add_gather(xv, idxv, pev):
    mesh = plsc.VectorSubcoreMesh(
        core_axis_name="c", subcore_axis_name="s",
        num_cores=NC, num_subcores=NS,
    )
    return pl.kernel(
        _body,
        out_type=jax.ShapeDtypeStruct((XROWS, 128), jnp.float32),
        mesh=mesh,
        scratch_types=[
            pltpu.VMEM((BATCH, 128), jnp.int32),
            pltpu.VMEM((NBUF, CROWS), jnp.int32),
            pltpu.VMEM((NBUF, CROWS, 128), jnp.float32),
            pltpu.SemaphoreType.DMA((NBUF,)),
            pltpu.SemaphoreType.DMA((NBUF,)),
            pltpu.SemaphoreType.DMA((NBUF,)),
        ],
    )(xv, idxv, pev)


def kernel(x, index, pe):
    xv = jnp.transpose(x.reshape(SEQ_LEN, BATCH, 8, 128),
                       (0, 2, 1, 3)).reshape(XROWS, 128)
    pev = pe.reshape(PEROWS, 128)
    idxv = index.astype(jnp.int32).T.reshape(128, 128)
    o = _sc_add_gather(xv, idxv, pev)
    return jnp.transpose(o.reshape(SEQ_LEN, 8, BATCH, 128),
                         (0, 2, 1, 3)).reshape(SEQ_LEN, BATCH, D_MODEL)
